# baseline (speedup 1.0000x reference)
import functools

import jax
import jax.numpy as jnp
from jax import lax
from jax.experimental import pallas as pl
from jax.experimental.pallas import tpu as pltpu
from jax.experimental.pallas import tpu_sc as plsc

N_NODES = 10000
N_EDGES = 320000

_NCORES = 2
_NSUB = 16
_CHUNK = 80


def _make_segsum(n, e, d, nbuf):
    nw = _NCORES * _NSUB
    epw = e // nw
    nch = epw // _CHUNK
    rps = (n // _NSUB) // 8 * 8
    tail = n - rps * _NSUB
    assert tail % 8 == 0

    mesh = plsc.VectorSubcoreMesh(core_axis_name="c", subcore_axis_name="s")

    @functools.partial(
        pl.kernel,
        out_type=jax.ShapeDtypeStruct((_NCORES * n, d), jnp.float32),
        mesh=mesh,
        compiler_params=pltpu.CompilerParams(use_tc_tiling_on_sc=False,
                                             skip_device_barrier=True),
        scratch_types=[
            pltpu.VMEM((nch, _CHUNK), jnp.int32),
            pltpu.VMEM((nch, _CHUNK), jnp.int32),
            pltpu.VMEM((nbuf, _CHUNK, d), jnp.float32),
            pltpu.VMEM_SHARED((n, d), jnp.float32),
            pltpu.SemaphoreType.DMA((nbuf,)),
        ],
    )
    def segsum(x_hbm, src_hbm, dst_hbm, zeros_hbm, out_hbm,
               srcbuf, dstbuf, rows, acc, sems):
        c = lax.axis_index("c")
        s = lax.axis_index("s")
        w = c * _NSUB + s
        pltpu.sync_copy(zeros_hbm.at[pl.ds(0, rps)],
                        acc.at[pl.ds(s * rps, rps)])

        @pl.when(s == 0)
        def _():
            pltpu.sync_copy(zeros_hbm.at[pl.ds(0, tail)],
                            acc.at[pl.ds(rps * _NSUB, tail)])

        pltpu.sync_copy(src_hbm.at[w], srcbuf)
        pltpu.sync_copy(dst_hbm.at[w], dstbuf)
        plsc.subcore_barrier()

        for b in range(nbuf):
            pltpu.async_copy(x_hbm.at[srcbuf.at[b]], rows.at[b], sems.at[b])

        @pl.loop(0, nch, step=nbuf)
        def _(g0):
            for b in range(nbuf):
                g = g0 + b

                @pl.when(g < nch)
                def _():
                    pltpu.make_async_copy(x_hbm.at[srcbuf.at[g]], rows.at[b],
                                          sems.at[b]).wait()
                    pltpu.sync_copy(rows.at[b], acc.at[dstbuf.at[g]],
                                    add=True)
                    nxt = g + nbuf

                    @pl.when(nxt < nch)
                    def _():
                        pltpu.async_copy(x_hbm.at[srcbuf.at[nxt]],
                                         rows.at[b], sems.at[b])

        plsc.subcore_barrier()
        pltpu.sync_copy(acc.at[pl.ds(s * rps, rps)],
                        out_hbm.at[pl.ds(c * n + s * rps, rps)])

        @pl.when(s == 0)
        def _():
            pltpu.sync_copy(acc.at[pl.ds(rps * _NSUB, tail)],
                            out_hbm.at[pl.ds(c * n + rps * _NSUB, tail)])

    return segsum


_segsum128 = _make_segsum(N_NODES, N_EDGES, 128, 3)
_segsum64 = _make_segsum(N_NODES, N_EDGES, 64, 6)


def _make_mlp(n, din, dh, dout, with_relu_out, block):
    grid = n // block

    nblk = n // block

    def row_spec(d):
        return pl.BlockSpec((block, d), lambda i: (i, 0))

    def half_spec(d, half):
        return pl.BlockSpec((block, d), lambda i, h=half: (i + h * nblk, 0))

    def full_spec(r, c):
        return pl.BlockSpec((r, c), lambda i: (0, 0))

    out_shapes = [jax.ShapeDtypeStruct((n, dout), jnp.float32)]
    out_specs = [row_spec(dout)]
    if with_relu_out:
        out_shapes.append(jax.ShapeDtypeStruct((n, dout), jnp.float32))
        out_specs.append(row_spec(dout))

    def body(eps_ref, x_ref, a0_ref, a1_ref, wa_ref, ba_ref, wb_ref, bb_ref,
             o0_ref, *rest):
        h = (1.0 + eps_ref[0]) * x_ref[...] + a0_ref[...] + a1_ref[...]
        t = jnp.maximum(
            jnp.dot(h, wa_ref[...], preferred_element_type=jnp.float32)
            + ba_ref[...], 0.0)
        o = jnp.dot(t, wb_ref[...], preferred_element_type=jnp.float32) \
            + bb_ref[...]
        o0_ref[...] = o
        if with_relu_out:
            rest[0][...] = jnp.maximum(o, 0.0)

    return pl.pallas_call(
        body,
        grid=(grid,),
        in_specs=[
            pl.BlockSpec(memory_space=pltpu.SMEM),
            row_spec(din), half_spec(din, 0), half_spec(din, 1),
            full_spec(din, dh), full_spec(1, dh),
            full_spec(dh, dout), full_spec(1, dout),
        ],
        out_specs=out_specs,
        out_shape=out_shapes,
    )


_mlp1 = _make_mlp(N_NODES, 128, 64, 64, True, 2000)
_mlp2 = _make_mlp(N_NODES, 64, 64, 64, False, 2000)


def kernel(x, W1a, b1a, W1b, b1b, eps1, W2a, b2a, W2b, b2b, eps2, edge_index):
    n = x.shape[0]
    e = edge_index.shape[1]
    nw = _NCORES * _NSUB
    src = jnp.reshape(edge_index[0], (nw, e // (nw * _CHUNK), _CHUNK))
    dst = jnp.reshape(edge_index[1], (nw, e // (nw * _CHUNK), _CHUNK))
    rps = (n // _NSUB) // 8 * 8
    z128 = jnp.zeros((rps, 128), jnp.float32)
    z64 = jnp.zeros((rps, 64), jnp.float32)

    agg1 = _segsum128(x, src, dst, z128)
    eps1v = jnp.reshape(eps1, (1,))
    emb, h2 = _mlp1(eps1v, x, agg1, agg1,
                    W1a, jnp.reshape(b1a, (1, -1)),
                    W1b, jnp.reshape(b1b, (1, -1)))

    agg2 = _segsum64(h2, src, dst, z64)
    eps2v = jnp.reshape(eps2, (1,))
    (logits,) = _mlp2(eps2v, h2, agg2, agg2,
                      W2a, jnp.reshape(b2a, (1, -1)),
                      W2b, jnp.reshape(b2b, (1, -1)))
    return (logits, emb)


# speedup vs baseline: 1.0010x; 1.0010x over previous
import functools

import jax
import jax.numpy as jnp
from jax import lax
from jax.experimental import pallas as pl
from jax.experimental.pallas import tpu as pltpu
from jax.experimental.pallas import tpu_sc as plsc

N_NODES = 10000
N_EDGES = 320000

_NCORES = 2
_NSUB = 16
_CHUNK = 80


def _make_segsum(n, e, d, nbuf, tc_tiling):
    nw = _NCORES * _NSUB
    epw = e // nw
    nch = epw // _CHUNK
    rps = (n // _NSUB) // 8 * 8
    tail = n - rps * _NSUB
    assert tail % 8 == 0

    mesh = plsc.VectorSubcoreMesh(core_axis_name="c", subcore_axis_name="s")

    @functools.partial(
        pl.kernel,
        out_type=jax.ShapeDtypeStruct((_NCORES * n, d), jnp.float32),
        mesh=mesh,
        compiler_params=pltpu.CompilerParams(use_tc_tiling_on_sc=tc_tiling),
        scratch_types=[
            pltpu.VMEM((nch, _CHUNK), jnp.int32),
            pltpu.VMEM((nch, _CHUNK), jnp.int32),
            pltpu.VMEM((nbuf, _CHUNK, d), jnp.float32),
            pltpu.VMEM_SHARED((n, d), jnp.float32),
            pltpu.SemaphoreType.DMA((nbuf,)),
        ],
    )
    def segsum(x_hbm, src_hbm, dst_hbm, zeros_hbm, out_hbm,
               srcbuf, dstbuf, rows, acc, sems):
        c = lax.axis_index("c")
        s = lax.axis_index("s")
        w = c * _NSUB + s
        pltpu.sync_copy(zeros_hbm.at[pl.ds(0, rps)],
                        acc.at[pl.ds(s * rps, rps)])

        @pl.when(s == 0)
        def _():
            pltpu.sync_copy(zeros_hbm.at[pl.ds(0, tail)],
                            acc.at[pl.ds(rps * _NSUB, tail)])

        pltpu.sync_copy(src_hbm.at[w], srcbuf)
        pltpu.sync_copy(dst_hbm.at[w], dstbuf)
        plsc.subcore_barrier()

        for b in range(nbuf):
            pltpu.async_copy(x_hbm.at[srcbuf.at[b]], rows.at[b], sems.at[b])

        @pl.loop(0, nch, step=nbuf)
        def _(g0):
            for b in range(nbuf):
                g = g0 + b

                @pl.when(g < nch)
                def _():
                    pltpu.make_async_copy(x_hbm.at[srcbuf.at[g]], rows.at[b],
                                          sems.at[b]).wait()
                    pltpu.sync_copy(rows.at[b], acc.at[dstbuf.at[g]],
                                    add=True)
                    nxt = g + nbuf

                    @pl.when(nxt < nch)
                    def _():
                        pltpu.async_copy(x_hbm.at[srcbuf.at[nxt]],
                                         rows.at[b], sems.at[b])

        plsc.subcore_barrier()
        pltpu.sync_copy(acc.at[pl.ds(s * rps, rps)],
                        out_hbm.at[pl.ds(c * n + s * rps, rps)])

        @pl.when(s == 0)
        def _():
            pltpu.sync_copy(acc.at[pl.ds(rps * _NSUB, tail)],
                            out_hbm.at[pl.ds(c * n + rps * _NSUB, tail)])

    return segsum


_segsum128 = _make_segsum(N_NODES, N_EDGES, 128, 3, False)
_segsum64 = _make_segsum(N_NODES, N_EDGES, 64, 6, False)


def _make_mlp(n, din, dh, dout, with_relu_out, block):
    grid = n // block

    nblk = n // block

    def row_spec(d):
        return pl.BlockSpec((block, d), lambda i: (i, 0))

    def half_spec(d, half):
        return pl.BlockSpec((block, d), lambda i, h=half: (i + h * nblk, 0))

    def full_spec(r, c):
        return pl.BlockSpec((r, c), lambda i: (0, 0))

    out_shapes = [jax.ShapeDtypeStruct((n, dout), jnp.float32)]
    out_specs = [row_spec(dout)]
    if with_relu_out:
        out_shapes.append(jax.ShapeDtypeStruct((n, dout), jnp.float32))
        out_specs.append(row_spec(dout))

    def body(eps_ref, x_ref, a0_ref, a1_ref, wa_ref, ba_ref, wb_ref, bb_ref,
             o0_ref, *rest):
        h = (1.0 + eps_ref[0]) * x_ref[...] + a0_ref[...] + a1_ref[...]
        t = jnp.maximum(
            jnp.dot(h, wa_ref[...], preferred_element_type=jnp.float32)
            + ba_ref[...], 0.0)
        o = jnp.dot(t, wb_ref[...], preferred_element_type=jnp.float32) \
            + bb_ref[...]
        o0_ref[...] = o
        if with_relu_out:
            rest[0][...] = jnp.maximum(o, 0.0)

    return pl.pallas_call(
        body,
        grid=(grid,),
        in_specs=[
            pl.BlockSpec(memory_space=pltpu.SMEM),
            row_spec(din), half_spec(din, 0), half_spec(din, 1),
            full_spec(din, dh), full_spec(1, dh),
            full_spec(dh, dout), full_spec(1, dout),
        ],
        out_specs=out_specs,
        out_shape=out_shapes,
    )


_mlp1 = _make_mlp(N_NODES, 128, 64, 64, True, 2000)
_mlp2 = _make_mlp(N_NODES, 64, 64, 64, False, 2000)


def kernel(x, W1a, b1a, W1b, b1b, eps1, W2a, b2a, W2b, b2b, eps2, edge_index):
    n = x.shape[0]
    e = edge_index.shape[1]
    nw = _NCORES * _NSUB
    src = jnp.reshape(edge_index[0], (nw, e // (nw * _CHUNK), _CHUNK))
    dst = jnp.reshape(edge_index[1], (nw, e // (nw * _CHUNK), _CHUNK))
    rps = (n // _NSUB) // 8 * 8
    z128 = jnp.zeros((rps, 128), jnp.float32)
    z64 = jnp.zeros((rps, 64), jnp.float32)

    agg1 = _segsum128(x, src, dst, z128)
    eps1v = jnp.reshape(eps1, (1,))
    emb, h2 = _mlp1(eps1v, x, agg1, agg1,
                    W1a, jnp.reshape(b1a, (1, -1)),
                    W1b, jnp.reshape(b1b, (1, -1)))

    agg2 = _segsum64(h2, src, dst, z64)
    eps2v = jnp.reshape(eps2, (1,))
    (logits,) = _mlp2(eps2v, h2, agg2, agg2,
                      W2a, jnp.reshape(b2a, (1, -1)),
                      W2b, jnp.reshape(b2b, (1, -1)))
    return (logits, emb)


# speedup vs baseline: 1.0039x; 1.0028x over previous
import functools

import jax
import jax.numpy as jnp
from jax import lax
from jax.experimental import pallas as pl
from jax.experimental.pallas import tpu as pltpu
from jax.experimental.pallas import tpu_sc as plsc

N_NODES = 10000
N_EDGES = 320000

_NCORES = 2
_NSUB = 16
_CHUNK = 80


def _make_segsum(n, e, d, nbuf, tc_tiling):
    nw = _NCORES * _NSUB
    epw = e // nw
    nch = epw // _CHUNK
    rps = (n // _NSUB) // 8 * 8
    tail = n - rps * _NSUB
    assert tail % 8 == 0

    mesh = plsc.VectorSubcoreMesh(core_axis_name="c", subcore_axis_name="s")

    @functools.partial(
        pl.kernel,
        out_type=jax.ShapeDtypeStruct((_NCORES * n, d), jnp.float32),
        mesh=mesh,
        compiler_params=pltpu.CompilerParams(use_tc_tiling_on_sc=tc_tiling),
        scratch_types=[
            pltpu.VMEM((epw,), jnp.int32),
            pltpu.VMEM((epw,), jnp.int32),
            pltpu.VMEM((nbuf, _CHUNK, d), jnp.float32),
            pltpu.VMEM_SHARED((n, d), jnp.float32),
            pltpu.SemaphoreType.DMA((nbuf,)),
        ],
    )
    def segsum(x_hbm, src_hbm, dst_hbm, zeros_hbm, out_hbm,
               srcbuf, dstbuf, rows, acc, sems):
        c = lax.axis_index("c")
        s = lax.axis_index("s")
        w = c * _NSUB + s
        pltpu.sync_copy(zeros_hbm.at[pl.ds(0, rps)],
                        acc.at[pl.ds(s * rps, rps)])

        @pl.when(s == 0)
        def _():
            pltpu.sync_copy(zeros_hbm.at[pl.ds(0, tail)],
                            acc.at[pl.ds(rps * _NSUB, tail)])

        base = pl.multiple_of(w * epw, 8)
        pltpu.sync_copy(src_hbm.at[pl.ds(base, epw)], srcbuf)
        pltpu.sync_copy(dst_hbm.at[pl.ds(base, epw)], dstbuf)
        plsc.subcore_barrier()

        def src_idx(g):
            return srcbuf.at[pl.ds(pl.multiple_of(g * _CHUNK, 8), _CHUNK)]

        def dst_idx(g):
            return dstbuf.at[pl.ds(pl.multiple_of(g * _CHUNK, 8), _CHUNK)]

        for b in range(nbuf):
            pltpu.async_copy(x_hbm.at[src_idx(b)], rows.at[b], sems.at[b])

        @pl.loop(0, nch, step=nbuf)
        def _(g0):
            for b in range(nbuf):
                g = g0 + b

                @pl.when(g < nch)
                def _():
                    pltpu.make_async_copy(x_hbm.at[src_idx(g)], rows.at[b],
                                          sems.at[b]).wait()
                    pltpu.sync_copy(rows.at[b], acc.at[dst_idx(g)],
                                    add=True)
                    nxt = g + nbuf

                    @pl.when(nxt < nch)
                    def _():
                        pltpu.async_copy(x_hbm.at[src_idx(nxt)],
                                         rows.at[b], sems.at[b])

        plsc.subcore_barrier()
        pltpu.sync_copy(acc.at[pl.ds(s * rps, rps)],
                        out_hbm.at[pl.ds(c * n + s * rps, rps)])

        @pl.when(s == 0)
        def _():
            pltpu.sync_copy(acc.at[pl.ds(rps * _NSUB, tail)],
                            out_hbm.at[pl.ds(c * n + rps * _NSUB, tail)])

    return segsum


_segsum128 = _make_segsum(N_NODES, N_EDGES, 128, 3, False)
_segsum64 = _make_segsum(N_NODES, N_EDGES, 64, 6, False)


def _make_mlp(n, din, dh, dout, with_relu_out, block):
    grid = n // block

    nblk = n // block

    def row_spec(d):
        return pl.BlockSpec((block, d), lambda i: (i, 0))

    def half_spec(d, half):
        return pl.BlockSpec((block, d), lambda i, h=half: (i + h * nblk, 0))

    def full_spec(r, c):
        return pl.BlockSpec((r, c), lambda i: (0, 0))

    out_shapes = [jax.ShapeDtypeStruct((n, dout), jnp.float32)]
    out_specs = [row_spec(dout)]
    if with_relu_out:
        out_shapes.append(jax.ShapeDtypeStruct((n, dout), jnp.float32))
        out_specs.append(row_spec(dout))

    def body(eps_ref, x_ref, a0_ref, a1_ref, wa_ref, ba_ref, wb_ref, bb_ref,
             o0_ref, *rest):
        h = (1.0 + eps_ref[0]) * x_ref[...] + a0_ref[...] + a1_ref[...]
        t = jnp.maximum(
            jnp.dot(h, wa_ref[...], preferred_element_type=jnp.float32)
            + ba_ref[...], 0.0)
        o = jnp.dot(t, wb_ref[...], preferred_element_type=jnp.float32) \
            + bb_ref[...]
        o0_ref[...] = o
        if with_relu_out:
            rest[0][...] = jnp.maximum(o, 0.0)

    return pl.pallas_call(
        body,
        grid=(grid,),
        in_specs=[
            pl.BlockSpec(memory_space=pltpu.SMEM),
            row_spec(din), half_spec(din, 0), half_spec(din, 1),
            full_spec(din, dh), full_spec(1, dh),
            full_spec(dh, dout), full_spec(1, dout),
        ],
        out_specs=out_specs,
        out_shape=out_shapes,
    )


_mlp1 = _make_mlp(N_NODES, 128, 64, 64, True, 2000)
_mlp2 = _make_mlp(N_NODES, 64, 64, 64, False, 2000)


def kernel(x, W1a, b1a, W1b, b1b, eps1, W2a, b2a, W2b, b2b, eps2, edge_index):
    n = x.shape[0]
    src = edge_index[0]
    dst = edge_index[1]
    rps = (n // _NSUB) // 8 * 8
    z128 = jnp.zeros((rps, 128), jnp.float32)
    z64 = jnp.zeros((rps, 64), jnp.float32)

    agg1 = _segsum128(x, src, dst, z128)
    eps1v = jnp.reshape(eps1, (1,))
    emb, h2 = _mlp1(eps1v, x, agg1, agg1,
                    W1a, jnp.reshape(b1a, (1, -1)),
                    W1b, jnp.reshape(b1b, (1, -1)))

    agg2 = _segsum64(h2, src, dst, z64)
    eps2v = jnp.reshape(eps2, (1,))
    (logits,) = _mlp2(eps2v, h2, agg2, agg2,
                      W2a, jnp.reshape(b2a, (1, -1)),
                      W2b, jnp.reshape(b2b, (1, -1)))
    return (logits, emb)


# speedup vs baseline: 1.1220x; 1.1177x over previous
import functools

import jax
import jax.numpy as jnp
from jax import lax
from jax.experimental import pallas as pl
from jax.experimental.pallas import tpu as pltpu
from jax.experimental.pallas import tpu_sc as plsc

N_NODES = 10000
N_EDGES = 320000

_NCORES = 2
_NSUB = 16
_CHUNK = 80


def _make_segsum(n, e, d, nbuf, tc_tiling):
    nw = _NCORES * _NSUB
    epw = e // nw
    nch = epw // _CHUNK
    rps = (n // _NSUB) // 8 * 8
    tail = n - rps * _NSUB
    assert tail % 8 == 0

    mesh = plsc.VectorSubcoreMesh(core_axis_name="c", subcore_axis_name="s")

    @functools.partial(
        pl.kernel,
        out_type=jax.ShapeDtypeStruct((_NCORES * n, d), jnp.float32),
        mesh=mesh,
        compiler_params=pltpu.CompilerParams(use_tc_tiling_on_sc=tc_tiling),
        scratch_types=[
            pltpu.VMEM((epw,), jnp.int32),
            pltpu.VMEM((epw,), jnp.int32),
            pltpu.VMEM((nbuf, _CHUNK, d), jnp.float32),
            pltpu.VMEM_SHARED((n, d), jnp.float32),
            pltpu.SemaphoreType.DMA((nbuf,)),
        ],
    )
    def segsum(x_hbm, src_hbm, dst_hbm, zeros_hbm, out_hbm,
               srcbuf, dstbuf, rows, acc, sems):
        c = lax.axis_index("c")
        s = lax.axis_index("s")
        w = c * _NSUB + s
        pltpu.sync_copy(zeros_hbm.at[pl.ds(0, rps)],
                        acc.at[pl.ds(s * rps, rps)])

        @pl.when(s == 0)
        def _():
            pltpu.sync_copy(zeros_hbm.at[pl.ds(0, tail)],
                            acc.at[pl.ds(rps * _NSUB, tail)])

        base = pl.multiple_of(w * epw, 8)
        pltpu.sync_copy(src_hbm.at[pl.ds(base, epw)], srcbuf)
        pltpu.sync_copy(dst_hbm.at[pl.ds(base, epw)], dstbuf)
        plsc.subcore_barrier()

        def src_idx(g):
            return srcbuf.at[pl.ds(pl.multiple_of(g * _CHUNK, 8), _CHUNK)]

        def dst_idx(g):
            return dstbuf.at[pl.ds(pl.multiple_of(g * _CHUNK, 8), _CHUNK)]

        for b in range(nbuf):
            pltpu.async_copy(x_hbm.at[src_idx(b)], rows.at[b], sems.at[b])

        @pl.loop(0, nch, step=nbuf)
        def _(g0):
            for b in range(nbuf):
                g = g0 + b

                @pl.when(g < nch)
                def _():
                    pltpu.make_async_copy(x_hbm.at[src_idx(g)], rows.at[b],
                                          sems.at[b]).wait()
                    pltpu.sync_copy(rows.at[b], acc.at[dst_idx(g)],
                                    add=True)
                    nxt = g + nbuf

                    @pl.when(nxt < nch)
                    def _():
                        pltpu.async_copy(x_hbm.at[src_idx(nxt)],
                                         rows.at[b], sems.at[b])

        plsc.subcore_barrier()
        pltpu.sync_copy(acc.at[pl.ds(s * rps, rps)],
                        out_hbm.at[pl.ds(c * n + s * rps, rps)])

        @pl.when(s == 0)
        def _():
            pltpu.sync_copy(acc.at[pl.ds(rps * _NSUB, tail)],
                            out_hbm.at[pl.ds(c * n + rps * _NSUB, tail)])

    return segsum


_segsum64 = _make_segsum(N_NODES, N_EDGES, 64, 6, False)

_BLOCK = 2000
_NBLK = N_NODES // _BLOCK


def _row_spec(d):
    return pl.BlockSpec((_BLOCK, d), lambda i: (i, 0))


def _half_spec(d, half):
    return pl.BlockSpec((_BLOCK, d), lambda i, h=half: (i + h * _NBLK, 0))


def _full_spec(r, c):
    return pl.BlockSpec((r, c), lambda i: (0, 0))


def _proj_body(x_ref, w_ref, v_ref):
    v_ref[...] = jnp.dot(x_ref[...], w_ref[...],
                         preferred_element_type=jnp.float32)


_proj = pl.pallas_call(
    _proj_body,
    grid=(_NBLK,),
    in_specs=[_row_spec(128), _full_spec(128, 64)],
    out_specs=_row_spec(64),
    out_shape=jax.ShapeDtypeStruct((N_NODES, 64), jnp.float32),
)


def _mid_body(eps_ref, v_ref, a0_ref, a1_ref, ba_ref, wb_ref, bb_ref,
              w2a_ref, emb_ref, u_ref):
    t = jnp.maximum((1.0 + eps_ref[0]) * v_ref[...] + a0_ref[...]
                    + a1_ref[...] + ba_ref[...], 0.0)
    emb = jnp.dot(t, wb_ref[...], preferred_element_type=jnp.float32) \
        + bb_ref[...]
    emb_ref[...] = emb
    h2 = jnp.maximum(emb, 0.0)
    u_ref[...] = jnp.dot(h2, w2a_ref[...], preferred_element_type=jnp.float32)


_mid = pl.pallas_call(
    _mid_body,
    grid=(_NBLK,),
    in_specs=[
        pl.BlockSpec(memory_space=pltpu.SMEM),
        _row_spec(64), _half_spec(64, 0), _half_spec(64, 1),
        _full_spec(1, 64), _full_spec(64, 64), _full_spec(1, 64),
        _full_spec(64, 64),
    ],
    out_specs=[_row_spec(64), _row_spec(64)],
    out_shape=[jax.ShapeDtypeStruct((N_NODES, 64), jnp.float32),
               jax.ShapeDtypeStruct((N_NODES, 64), jnp.float32)],
)


def _out_body(eps_ref, u_ref, a0_ref, a1_ref, ba_ref, wb_ref, bb_ref,
              o_ref):
    t = jnp.maximum((1.0 + eps_ref[0]) * u_ref[...] + a0_ref[...]
                    + a1_ref[...] + ba_ref[...], 0.0)
    o_ref[...] = jnp.dot(t, wb_ref[...], preferred_element_type=jnp.float32) \
        + bb_ref[...]


_out = pl.pallas_call(
    _out_body,
    grid=(_NBLK,),
    in_specs=[
        pl.BlockSpec(memory_space=pltpu.SMEM),
        _row_spec(64), _half_spec(64, 0), _half_spec(64, 1),
        _full_spec(1, 64), _full_spec(64, 64), _full_spec(1, 64),
    ],
    out_specs=_row_spec(64),
    out_shape=jax.ShapeDtypeStruct((N_NODES, 64), jnp.float32),
)


def kernel(x, W1a, b1a, W1b, b1b, eps1, W2a, b2a, W2b, b2b, eps2, edge_index):
    n = x.shape[0]
    src = edge_index[0]
    dst = edge_index[1]
    rps = (n // _NSUB) // 8 * 8
    z64 = jnp.zeros((rps, 64), jnp.float32)

    v = _proj(x, W1a)
    aggv = _segsum64(v, src, dst, z64)
    eps1v = jnp.reshape(eps1, (1,))
    emb, u = _mid(eps1v, v, aggv, aggv,
                  jnp.reshape(b1a, (1, -1)), W1b,
                  jnp.reshape(b1b, (1, -1)), W2a)

    aggu = _segsum64(u, src, dst, z64)
    eps2v = jnp.reshape(eps2, (1,))
    logits = _out(eps2v, u, aggu, aggu,
                  jnp.reshape(b2a, (1, -1)), W2b,
                  jnp.reshape(b2b, (1, -1)))
    return (logits, emb)


# speedup vs baseline: 1.1821x; 1.0535x over previous
import functools

import jax
import jax.numpy as jnp
from jax import lax
from jax.experimental import pallas as pl
from jax.experimental.pallas import tpu as pltpu
from jax.experimental.pallas import tpu_sc as plsc

N_NODES = 10000
N_EDGES = 320000

_NCORES = 2
_NSUB = 16
_CHUNK = 80


def _make_segsum(n, e, d, nbuf, tc_tiling):
    nw = _NCORES * _NSUB
    epw = e // nw
    nch = epw // _CHUNK
    rps = (n // _NSUB) // 8 * 8
    tail = n - rps * _NSUB
    assert tail % 8 == 0

    mesh = plsc.VectorSubcoreMesh(core_axis_name="c", subcore_axis_name="s")

    @functools.partial(
        pl.kernel,
        out_type=jax.ShapeDtypeStruct((_NCORES * n, d), jnp.float32),
        mesh=mesh,
        compiler_params=pltpu.CompilerParams(use_tc_tiling_on_sc=tc_tiling),
        scratch_types=[
            pltpu.VMEM((epw,), jnp.int32),
            pltpu.VMEM((epw,), jnp.int32),
            pltpu.VMEM((nbuf, _CHUNK, d), jnp.float32),
            pltpu.VMEM_SHARED((n, d), jnp.float32),
            pltpu.SemaphoreType.DMA((nbuf,)),
        ],
    )
    def segsum(x_hbm, edges_hbm, zeros_hbm, out_hbm,
               srcbuf, dstbuf, rows, acc, sems):
        c = lax.axis_index("c")
        s = lax.axis_index("s")
        w = c * _NSUB + s
        pltpu.sync_copy(zeros_hbm.at[pl.ds(0, rps)],
                        acc.at[pl.ds(s * rps, rps)])

        @pl.when(s == 0)
        def _():
            pltpu.sync_copy(zeros_hbm.at[pl.ds(0, tail)],
                            acc.at[pl.ds(rps * _NSUB, tail)])

        base = pl.multiple_of(w * epw, 8)
        pltpu.sync_copy(edges_hbm.at[0, pl.ds(base, epw)], srcbuf)
        pltpu.sync_copy(edges_hbm.at[1, pl.ds(base, epw)], dstbuf)
        plsc.subcore_barrier()

        def src_idx(g):
            return srcbuf.at[pl.ds(pl.multiple_of(g * _CHUNK, 8), _CHUNK)]

        def dst_idx(g):
            return dstbuf.at[pl.ds(pl.multiple_of(g * _CHUNK, 8), _CHUNK)]

        for b in range(nbuf):
            pltpu.async_copy(x_hbm.at[src_idx(b)], rows.at[b], sems.at[b])

        @pl.loop(0, nch, step=nbuf)
        def _(g0):
            for b in range(nbuf):
                g = g0 + b

                @pl.when(g < nch)
                def _():
                    pltpu.make_async_copy(x_hbm.at[src_idx(g)], rows.at[b],
                                          sems.at[b]).wait()
                    pltpu.sync_copy(rows.at[b], acc.at[dst_idx(g)],
                                    add=True)
                    nxt = g + nbuf

                    @pl.when(nxt < nch)
                    def _():
                        pltpu.async_copy(x_hbm.at[src_idx(nxt)],
                                         rows.at[b], sems.at[b])

        plsc.subcore_barrier()
        pltpu.sync_copy(acc.at[pl.ds(s * rps, rps)],
                        out_hbm.at[pl.ds(c * n + s * rps, rps)])

        @pl.when(s == 0)
        def _():
            pltpu.sync_copy(acc.at[pl.ds(rps * _NSUB, tail)],
                            out_hbm.at[pl.ds(c * n + rps * _NSUB, tail)])

    return segsum


_segsum64 = _make_segsum(N_NODES, N_EDGES, 64, 8, False)

_BLOCK = 2000
_NBLK = N_NODES // _BLOCK


def _row_spec(d):
    return pl.BlockSpec((_BLOCK, d), lambda i: (i, 0))


def _half_spec(d, half):
    return pl.BlockSpec((_BLOCK, d), lambda i, h=half: (i + h * _NBLK, 0))


def _full_spec(r, c):
    return pl.BlockSpec((r, c), lambda i: (0, 0))


def _proj_body(x_ref, w_ref, v_ref):
    v_ref[...] = jnp.dot(x_ref[...], w_ref[...],
                         preferred_element_type=jnp.float32)


_proj = pl.pallas_call(
    _proj_body,
    grid=(_NBLK,),
    in_specs=[_row_spec(128), _full_spec(128, 64)],
    out_specs=_row_spec(64),
    out_shape=jax.ShapeDtypeStruct((N_NODES, 64), jnp.float32),
)


def _mid_body(eps_ref, v_ref, a0_ref, a1_ref, ba_ref, wb_ref, bb_ref,
              w2a_ref, emb_ref, u_ref):
    t = jnp.maximum((1.0 + eps_ref[0]) * v_ref[...] + a0_ref[...]
                    + a1_ref[...] + ba_ref[...], 0.0)
    emb = jnp.dot(t, wb_ref[...], preferred_element_type=jnp.float32) \
        + bb_ref[...]
    emb_ref[...] = emb
    h2 = jnp.maximum(emb, 0.0)
    u_ref[...] = jnp.dot(h2, w2a_ref[...], preferred_element_type=jnp.float32)


_mid = pl.pallas_call(
    _mid_body,
    grid=(_NBLK,),
    in_specs=[
        pl.BlockSpec(memory_space=pltpu.SMEM),
        _row_spec(64), _half_spec(64, 0), _half_spec(64, 1),
        _full_spec(1, 64), _full_spec(64, 64), _full_spec(1, 64),
        _full_spec(64, 64),
    ],
    out_specs=[_row_spec(64), _row_spec(64)],
    out_shape=[jax.ShapeDtypeStruct((N_NODES, 64), jnp.float32),
               jax.ShapeDtypeStruct((N_NODES, 64), jnp.float32)],
)


def _out_body(eps_ref, u_ref, a0_ref, a1_ref, ba_ref, wb_ref, bb_ref,
              o_ref):
    t = jnp.maximum((1.0 + eps_ref[0]) * u_ref[...] + a0_ref[...]
                    + a1_ref[...] + ba_ref[...], 0.0)
    o_ref[...] = jnp.dot(t, wb_ref[...], preferred_element_type=jnp.float32) \
        + bb_ref[...]


_out = pl.pallas_call(
    _out_body,
    grid=(_NBLK,),
    in_specs=[
        pl.BlockSpec(memory_space=pltpu.SMEM),
        _row_spec(64), _half_spec(64, 0), _half_spec(64, 1),
        _full_spec(1, 64), _full_spec(64, 64), _full_spec(1, 64),
    ],
    out_specs=_row_spec(64),
    out_shape=jax.ShapeDtypeStruct((N_NODES, 64), jnp.float32),
)


def kernel(x, W1a, b1a, W1b, b1b, eps1, W2a, b2a, W2b, b2b, eps2, edge_index):
    n = x.shape[0]
    rps = (n // _NSUB) // 8 * 8
    z64 = jnp.zeros((rps, 64), jnp.float32)

    v = _proj(x, W1a)
    aggv = _segsum64(v, edge_index, z64)
    eps1v = jnp.reshape(eps1, (1,))
    emb, u = _mid(eps1v, v, aggv, aggv,
                  jnp.reshape(b1a, (1, -1)), W1b,
                  jnp.reshape(b1b, (1, -1)), W2a)

    aggu = _segsum64(u, edge_index, z64)
    eps2v = jnp.reshape(eps2, (1,))
    logits = _out(eps2v, u, aggu, aggu,
                  jnp.reshape(b2a, (1, -1)), W2b,
                  jnp.reshape(b2b, (1, -1)))
    return (logits, emb)


# speedup vs baseline: 1.1873x; 1.0044x over previous
import functools

import jax
import jax.numpy as jnp
from jax import lax
from jax.experimental import pallas as pl
from jax.experimental.pallas import tpu as pltpu
from jax.experimental.pallas import tpu_sc as plsc

N_NODES = 10000
N_EDGES = 320000

_NCORES = 2
_NSUB = 16
_CHUNK = 80


def _make_segsum(n, e, d, nbuf, tc_tiling):
    nw = _NCORES * _NSUB
    epw = e // nw
    nch = epw // _CHUNK
    rps = (n // _NSUB) // 8 * 8
    tail = n - rps * _NSUB
    assert tail % 8 == 0

    mesh = plsc.VectorSubcoreMesh(core_axis_name="c", subcore_axis_name="s")

    @functools.partial(
        pl.kernel,
        out_type=jax.ShapeDtypeStruct((_NCORES * n, d), jnp.float32),
        mesh=mesh,
        compiler_params=pltpu.CompilerParams(use_tc_tiling_on_sc=tc_tiling),
        scratch_types=[
            pltpu.VMEM((epw,), jnp.int32),
            pltpu.VMEM((epw,), jnp.int32),
            pltpu.VMEM((nbuf, _CHUNK, d), jnp.float32),
            pltpu.VMEM_SHARED((n, d), jnp.float32),
            pltpu.SemaphoreType.DMA((nbuf,)),
        ],
    )
    def segsum(x_hbm, edges_hbm, zeros_hbm, out_hbm,
               srcbuf, dstbuf, rows, acc, sems):
        c = lax.axis_index("c")
        s = lax.axis_index("s")
        w = c * _NSUB + s
        pltpu.sync_copy(zeros_hbm.at[pl.ds(0, rps)],
                        acc.at[pl.ds(s * rps, rps)])

        @pl.when(s == 0)
        def _():
            pltpu.sync_copy(zeros_hbm.at[pl.ds(0, tail)],
                            acc.at[pl.ds(rps * _NSUB, tail)])

        base = pl.multiple_of(w * epw, 8)
        pltpu.sync_copy(edges_hbm.at[0, pl.ds(base, epw)], srcbuf)
        pltpu.sync_copy(edges_hbm.at[1, pl.ds(base, epw)], dstbuf)
        plsc.subcore_barrier()

        def src_idx(g):
            return srcbuf.at[pl.ds(pl.multiple_of(g * _CHUNK, 8), _CHUNK)]

        def dst_idx(g):
            return dstbuf.at[pl.ds(pl.multiple_of(g * _CHUNK, 8), _CHUNK)]

        for b in range(nbuf):
            pltpu.async_copy(x_hbm.at[src_idx(b)], rows.at[b], sems.at[b])

        @pl.loop(0, nch, step=nbuf)
        def _(g0):
            for b in range(nbuf):
                g = g0 + b

                @pl.when(g < nch)
                def _():
                    pltpu.make_async_copy(x_hbm.at[src_idx(g)], rows.at[b],
                                          sems.at[b]).wait()
                    pltpu.sync_copy(rows.at[b], acc.at[dst_idx(g)],
                                    add=True)
                    nxt = g + nbuf

                    @pl.when(nxt < nch)
                    def _():
                        pltpu.async_copy(x_hbm.at[src_idx(nxt)],
                                         rows.at[b], sems.at[b])

        plsc.subcore_barrier()
        pltpu.sync_copy(acc.at[pl.ds(s * rps, rps)],
                        out_hbm.at[pl.ds(c * n + s * rps, rps)])

        @pl.when(s == 0)
        def _():
            pltpu.sync_copy(acc.at[pl.ds(rps * _NSUB, tail)],
                            out_hbm.at[pl.ds(c * n + rps * _NSUB, tail)])

    return segsum


_segsum64 = _make_segsum(N_NODES, N_EDGES, 64, 10, False)

_BLOCK = 2000
_NBLK = N_NODES // _BLOCK


def _row_spec(d):
    return pl.BlockSpec((_BLOCK, d), lambda i: (i, 0))


def _half_spec(d, half):
    return pl.BlockSpec((_BLOCK, d), lambda i, h=half: (i + h * _NBLK, 0))


def _full_spec(r, c):
    return pl.BlockSpec((r, c), lambda i: (0, 0))


def _proj_body(x_ref, w_ref, v_ref):
    v_ref[...] = jnp.dot(x_ref[...], w_ref[...],
                         preferred_element_type=jnp.float32)


_proj = pl.pallas_call(
    _proj_body,
    grid=(_NBLK,),
    in_specs=[_row_spec(128), _full_spec(128, 64)],
    out_specs=_row_spec(64),
    out_shape=jax.ShapeDtypeStruct((N_NODES, 64), jnp.float32),
)


def _mid_body(eps_ref, v_ref, a0_ref, a1_ref, ba_ref, wb_ref, bb_ref,
              w2a_ref, emb_ref, u_ref):
    t = jnp.maximum((1.0 + eps_ref[0]) * v_ref[...] + a0_ref[...]
                    + a1_ref[...] + ba_ref[...], 0.0)
    emb = jnp.dot(t, wb_ref[...], preferred_element_type=jnp.float32) \
        + bb_ref[...]
    emb_ref[...] = emb
    h2 = jnp.maximum(emb, 0.0)
    u_ref[...] = jnp.dot(h2, w2a_ref[...], preferred_element_type=jnp.float32)


_mid = pl.pallas_call(
    _mid_body,
    grid=(_NBLK,),
    in_specs=[
        pl.BlockSpec(memory_space=pltpu.SMEM),
        _row_spec(64), _half_spec(64, 0), _half_spec(64, 1),
        _full_spec(1, 64), _full_spec(64, 64), _full_spec(1, 64),
        _full_spec(64, 64),
    ],
    out_specs=[_row_spec(64), _row_spec(64)],
    out_shape=[jax.ShapeDtypeStruct((N_NODES, 64), jnp.float32),
               jax.ShapeDtypeStruct((N_NODES, 64), jnp.float32)],
)


def _out_body(eps_ref, u_ref, a0_ref, a1_ref, ba_ref, wb_ref, bb_ref,
              o_ref):
    t = jnp.maximum((1.0 + eps_ref[0]) * u_ref[...] + a0_ref[...]
                    + a1_ref[...] + ba_ref[...], 0.0)
    o_ref[...] = jnp.dot(t, wb_ref[...], preferred_element_type=jnp.float32) \
        + bb_ref[...]


_out = pl.pallas_call(
    _out_body,
    grid=(_NBLK,),
    in_specs=[
        pl.BlockSpec(memory_space=pltpu.SMEM),
        _row_spec(64), _half_spec(64, 0), _half_spec(64, 1),
        _full_spec(1, 64), _full_spec(64, 64), _full_spec(1, 64),
    ],
    out_specs=_row_spec(64),
    out_shape=jax.ShapeDtypeStruct((N_NODES, 64), jnp.float32),
)


def kernel(x, W1a, b1a, W1b, b1b, eps1, W2a, b2a, W2b, b2b, eps2, edge_index):
    n = x.shape[0]
    rps = (n // _NSUB) // 8 * 8
    z64 = jnp.zeros((rps, 64), jnp.float32)

    v = _proj(x, W1a)
    aggv = _segsum64(v, edge_index, z64)
    eps1v = jnp.reshape(eps1, (1,))
    emb, u = _mid(eps1v, v, aggv, aggv,
                  jnp.reshape(b1a, (1, -1)), W1b,
                  jnp.reshape(b1b, (1, -1)), W2a)

    aggu = _segsum64(u, edge_index, z64)
    eps2v = jnp.reshape(eps2, (1,))
    logits = _out(eps2v, u, aggu, aggu,
                  jnp.reshape(b2a, (1, -1)), W2b,
                  jnp.reshape(b2b, (1, -1)))
    return (logits, emb)
